# baseline (speedup 1.0000x reference)
import functools

import jax
import jax.numpy as jnp
from jax import lax
from jax.experimental import pallas as pl
from jax.experimental.pallas import tpu as pltpu
from jax.experimental.pallas import tpu_sc as plsc

N = 16384
F = 26
V = 100000
D = 64
S = 768

_INFO = plsc.get_sparse_core_info()
_NC = _INFO.num_cores
_NS = _INFO.num_subcores
_NW = _NC * _NS
_CW = N // _NW
_KB = 128
_NK = _CW // _KB


def _gather_body(cat_hbm, tab_hbm, out_hbm, idx_v, gidx_v, rows_v, sem):
    wid = lax.axis_index("s") * _NC + lax.axis_index("c")
    nbase = wid * _CW
    rbase = wid * _NK

    def field_step(f, carry):
        pltpu.sync_copy(cat_hbm.at[f, pl.ds(rbase, _NK)], idx_v)
        off = f * V
        for k in range(_NK):
            for j in range(_KB // 16):
                gidx_v[k, pl.ds(j * 16, 16)] = idx_v[k, pl.ds(j * 16, 16)] + off
        cps = [
            pltpu.async_copy(
                tab_hbm.at[gidx_v.at[k]],
                rows_v.at[pl.ds(k * _KB, _KB)],
                sem,
            )
            for k in range(_NK)
        ]
        for cp in cps:
            cp.wait()
        pltpu.sync_copy(
            rows_v, out_hbm.at[pl.ds(nbase, _CW), pl.ds(f * D, D)]
        )
        return carry

    lax.fori_loop(0, F, field_step, 0)


_gather = functools.partial(
    pl.kernel,
    out_type=jax.ShapeDtypeStruct((N, F * D), jnp.float32),
    mesh=plsc.VectorSubcoreMesh(core_axis_name="c", subcore_axis_name="s"),
    scratch_types=[
        pltpu.VMEM((_NK, _KB), jnp.int32),
        pltpu.VMEM((_NK, _KB), jnp.int32),
        pltpu.VMEM((_CW, D), jnp.float32),
        pltpu.SemaphoreType.DMA,
    ],
    compiler_params=pltpu.CompilerParams(use_tc_tiling_on_sc=False),
)(_gather_body)


_BN = 512


def _proj_body(sent_ref, g_ref, w_ref, b_ref, out_ref):
    acc = lax.dot_general(
        g_ref[...], w_ref[...],
        (((1,), (1,)), ((), ())),
        preferred_element_type=jnp.float32,
    )
    out_ref[:, :S] = sent_ref[...]
    out_ref[:, S:] = acc + b_ref[...]


def _project(sent, g, W, b2):
    return pl.pallas_call(
        _proj_body,
        grid=(N // _BN,),
        in_specs=[
            pl.BlockSpec((_BN, S), lambda i: (i, 0)),
            pl.BlockSpec((_BN, F * D), lambda i: (i, 0)),
            pl.BlockSpec((S, F * D), lambda i: (0, 0)),
            pl.BlockSpec((1, S), lambda i: (0, 0)),
        ],
        out_specs=pl.BlockSpec((_BN, 2 * S), lambda i: (i, 0)),
        out_shape=jax.ShapeDtypeStruct((N, 2 * S), jnp.float32),
    )(sent, g, W, b2)


def kernel(sentence_embeddings, categorical_data, tables, W, b):
    cat = categorical_data.astype(jnp.int32).reshape(F, N // _KB, _KB)
    tab = tables.reshape(F * V, D)
    g = _gather(cat, tab)
    return _project(sentence_embeddings, g, W, b.reshape(1, S))


# speedup vs baseline: 1.0010x; 1.0010x over previous
import functools

import jax
import jax.numpy as jnp
from jax import lax
from jax.experimental import pallas as pl
from jax.experimental.pallas import tpu as pltpu
from jax.experimental.pallas import tpu_sc as plsc

N = 16384
F = 26
V = 100000
D = 64
S = 768

_INFO = plsc.get_sparse_core_info()
_NC = _INFO.num_cores
_NS = _INFO.num_subcores
_NW = _NC * _NS
_CW = N // _NW
_KB = 128
_NK = _CW // _KB


def _gather_body(cat_hbm, tab_hbm, out_hbm, idx_v, rows_v, sem):
    wid = lax.axis_index("s") * _NC + lax.axis_index("c")
    nbase = wid * _CW

    def field_step(f, carry):
        pltpu.sync_copy(cat_hbm.at[f, pl.ds(nbase, _CW)], idx_v)
        cps = [
            pltpu.async_copy(
                tab_hbm.at[f].at[idx_v.at[pl.ds(k * _KB, _KB)]],
                rows_v.at[pl.ds(k * _KB, _KB)],
                sem,
            )
            for k in range(_NK)
        ]
        for cp in cps:
            cp.wait()
        pltpu.sync_copy(
            rows_v, out_hbm.at[pl.ds(nbase, _CW), pl.ds(f * D, D)]
        )
        return carry

    lax.fori_loop(0, F, field_step, 0)


_gather = functools.partial(
    pl.kernel,
    out_type=jax.ShapeDtypeStruct((N, F * D), jnp.float32),
    mesh=plsc.VectorSubcoreMesh(core_axis_name="c", subcore_axis_name="s"),
    scratch_types=[
        pltpu.VMEM((_CW,), jnp.int32),
        pltpu.VMEM((_CW, D), jnp.float32),
        pltpu.SemaphoreType.DMA,
    ],
    compiler_params=pltpu.CompilerParams(use_tc_tiling_on_sc=False),
)(_gather_body)


_BN = 512


def _proj_body(sent_ref, g_ref, w_ref, b_ref, out_ref):
    acc = lax.dot_general(
        g_ref[...], w_ref[...],
        (((1,), (1,)), ((), ())),
        preferred_element_type=jnp.float32,
    )
    out_ref[:, :S] = sent_ref[...]
    out_ref[:, S:] = acc + b_ref[...]


def _project(sent, g, W, b2):
    return pl.pallas_call(
        _proj_body,
        grid=(N // _BN,),
        in_specs=[
            pl.BlockSpec((_BN, S), lambda i: (i, 0)),
            pl.BlockSpec((_BN, F * D), lambda i: (i, 0)),
            pl.BlockSpec((S, F * D), lambda i: (0, 0)),
            pl.BlockSpec((1, S), lambda i: (0, 0)),
        ],
        out_specs=pl.BlockSpec((_BN, 2 * S), lambda i: (i, 0)),
        out_shape=jax.ShapeDtypeStruct((N, 2 * S), jnp.float32),
    )(sent, g, W, b2)


def kernel(sentence_embeddings, categorical_data, tables, W, b):
    cat = categorical_data.astype(jnp.int32)
    g = _gather(cat, tables)
    return _project(sentence_embeddings, g, W, b.reshape(1, S))
